# baseline (speedup 1.0000x reference)
import jax
import jax.numpy as jnp
from jax.experimental import pallas as pl

_VOCAB = 1000
_ROWS_PER_BLK = 512


def _onehot_block(x_ref, o_ref):
    idx = x_ref[0, 0, :]
    iota = jax.lax.broadcasted_iota(jnp.int32, (_ROWS_PER_BLK, _VOCAB), 1)
    o_ref[...] = (idx[:, None] == iota).astype(jnp.float32)


def kernel(x, W):
    B, H = x.shape
    N = B * H
    xf = x.reshape(N).astype(jnp.int32)
    G = N // _ROWS_PER_BLK
    x3 = xf.reshape(G, 1, _ROWS_PER_BLK)
    out = pl.pallas_call(
        _onehot_block,
        grid=(G,),
        in_specs=[pl.BlockSpec((1, 1, _ROWS_PER_BLK), lambda i: (i, 0, 0))],
        out_specs=pl.BlockSpec((_ROWS_PER_BLK, _VOCAB), lambda i: (i, 0)),
        out_shape=jax.ShapeDtypeStruct((N, _VOCAB), jnp.float32),
    )(x3)
    return out.reshape(B, H, _VOCAB)
